# baseline (speedup 1.0000x reference)
import functools

import jax
import jax.numpy as jnp
from jax import lax
from jax.experimental import pallas as pl
from jax.experimental.pallas import tpu as pltpu
from jax.experimental.pallas import tpu_sc as plsc

_MU = 3.5
_B = 16384
_D = 64
_NC = 2
_NS = 16
_NW = _NC * _NS
_BPW = _B // _NW
_CH = 128
_NCH = _BPW // _CH
_L = 16
_NG = _BPW // _L


def _sc_body(eu_hbm, ei_hbm, ub_hbm, ib_hbm, uidx_hbm, iidx_hbm, out_hbm,
             uidx_v, iidx_v, eu_v, ei_v, ub_v, ib_v, out_v, sem):
    wid = lax.axis_index("s") * _NC + lax.axis_index("c")
    base = wid * _BPW

    pltpu.sync_copy(uidx_hbm.at[pl.ds(base, _BPW)], uidx_v)
    pltpu.sync_copy(iidx_hbm.at[pl.ds(base, _BPW)], iidx_v)

    copies = []
    for j in range(_NCH):
        if _NCH == 1:
            usl, isl = uidx_v, iidx_v
            eud, eid, ubd, ibd = eu_v, ei_v, ub_v, ib_v
        else:
            sl = pl.ds(j * _CH, _CH)
            usl, isl = uidx_v.at[sl], iidx_v.at[sl]
            eud, eid, ubd, ibd = eu_v.at[sl], ei_v.at[sl], ub_v.at[sl], ib_v.at[sl]
        copies.append(pltpu.async_copy(eu_hbm.at[usl], eud, sem))
        copies.append(pltpu.async_copy(ei_hbm.at[isl], eid, sem))
        copies.append(pltpu.async_copy(ub_hbm.at[usl], ubd, sem))
        copies.append(pltpu.async_copy(ib_hbm.at[isl], ibd, sem))
    for c in copies:
        c.wait()

    lane = lax.iota(jnp.int32, _L)

    def group(g, carry):
        w = jnp.zeros((_L,), jnp.float32)
        for l in range(_L):
            r = g * _L + l
            acc = eu_v[r, pl.ds(0, _L)] * ei_v[r, pl.ds(0, _L)]
            for c in range(1, _D // _L):
                acc = acc + eu_v[r, pl.ds(c * _L, _L)] * ei_v[r, pl.ds(c * _L, _L)]
            s = jnp.sum(acc)
            w = jnp.where(lane == l, jnp.full((_L,), s), w)
        out_v[pl.ds(g * _L, _L)] = (
            w + ub_v[pl.ds(g * _L, _L)] + ib_v[pl.ds(g * _L, _L)] + _MU)
        return carry

    lax.fori_loop(0, _NG, group, 0)

    pltpu.sync_copy(out_v, out_hbm.at[pl.ds(base, _BPW)])


@jax.jit
def _sc_call(embed_user, embed_item, ub_flat, ib_flat, user_idx, item_idx):
    mesh = plsc.VectorSubcoreMesh(core_axis_name="c", subcore_axis_name="s")
    run = functools.partial(
        pl.kernel,
        mesh=mesh,
        compiler_params=pltpu.CompilerParams(
            needs_layout_passes=False, use_tc_tiling_on_sc=False),
        out_type=jax.ShapeDtypeStruct((_B,), jnp.float32),
        scratch_types=[
            pltpu.VMEM((_BPW,), jnp.int32),
            pltpu.VMEM((_BPW,), jnp.int32),
            pltpu.VMEM((_BPW, _D), jnp.float32),
            pltpu.VMEM((_BPW, _D), jnp.float32),
            pltpu.VMEM((_BPW,), jnp.float32),
            pltpu.VMEM((_BPW,), jnp.float32),
            pltpu.VMEM((_BPW,), jnp.float32),
            pltpu.SemaphoreType.DMA,
        ],
    )(_sc_body)
    return run(embed_user, embed_item, ub_flat, ib_flat, user_idx, item_idx)


def kernel(embed_user, embed_item, user_bias, item_bias, user_idx, item_idx):
    ub_flat = user_bias.reshape(-1)
    ib_flat = item_bias.reshape(-1)
    return _sc_call(embed_user, embed_item, ub_flat, ib_flat,
                    user_idx.astype(jnp.int32), item_idx.astype(jnp.int32))


# speedup vs baseline: 1.4744x; 1.4744x over previous
import functools

import jax
import jax.numpy as jnp
from jax import lax
from jax.experimental import pallas as pl
from jax.experimental.pallas import tpu as pltpu
from jax.experimental.pallas import tpu_sc as plsc

_MU = 3.5
_B = 16384
_D = 64
_NU = 1000000
_NI = 100000
_NC = 2
_NS = 16
_NW = _NC * _NS
_BPW = _B // _NW
_NP = 2
_BPC = _BPW // _NP
_L = 16
_NG = _BPC // _L


def _win(idx, n):
    return jnp.minimum(idx & -8, n - _L)


def _sc_body(eu_hbm, ei_hbm, ub_hbm, ib_hbm, uidx_hbm, iidx_hbm, out_hbm,
             uidx_v, iidx_v, eu_c, ei_c, ub_w, ib_w, out_v,
             sem, bsem):
    wid = lax.axis_index("s") * _NC + lax.axis_index("c")
    base = wid * _BPW

    pltpu.sync_copy(uidx_hbm.at[pl.ds(base, _BPW)], uidx_v)
    pltpu.sync_copy(iidx_hbm.at[pl.ds(base, _BPW)], iidx_v)

    lane = lax.iota(jnp.int32, _L)

    for p in range(_NP):
        p0 = p * _BPC

        def fire(g, carry):
            uv = uidx_v[pl.ds(p0 + g * _L, _L)]
            iv = iidx_v[pl.ds(p0 + g * _L, _L)]
            uwin = _win(uv, _NU)
            iwin = _win(iv, _NI)
            for l in range(_L):
                r = g * _L + l
                u = uv[l]
                i = iv[l]
                r16 = pl.multiple_of((p0 + r) * _L, 8)
                pltpu.async_copy(eu_hbm.at[pl.ds(u, 1), :],
                                 eu_c.at[pl.ds(r, 1), :], sem)
                pltpu.async_copy(ei_hbm.at[pl.ds(i, 1), :],
                                 ei_c.at[pl.ds(r, 1), :], sem)
                pltpu.async_copy(
                    ub_hbm.at[pl.ds(pl.multiple_of(uwin[l], 8), _L)],
                    ub_w.at[pl.ds(r16, _L)], bsem)
                pltpu.async_copy(
                    ib_hbm.at[pl.ds(pl.multiple_of(iwin[l], 8), _L)],
                    ib_w.at[pl.ds(r16, _L)], bsem)
            return carry

        lax.fori_loop(0, _NG, fire, 0)

        pltpu.make_async_copy(eu_hbm.at[pl.ds(0, _BPC), :], eu_c, sem).wait()
        pltpu.make_async_copy(ei_hbm.at[pl.ds(0, _BPC), :], ei_c, sem).wait()
        pltpu.make_async_copy(ub_hbm.at[pl.ds(0, _BPC * _L)],
                              ub_w.at[pl.ds(p0 * _L, _BPC * _L)], bsem).wait()
        pltpu.make_async_copy(ib_hbm.at[pl.ds(0, _BPC * _L)],
                              ib_w.at[pl.ds(p0 * _L, _BPC * _L)], bsem).wait()

        def group(g, carry):
            w = jnp.zeros((_L,), jnp.float32)
            for l in range(_L):
                r = g * _L + l
                acc = eu_c[r, pl.ds(0, _L)] * ei_c[r, pl.ds(0, _L)]
                for c in range(1, _D // _L):
                    acc = acc + (eu_c[r, pl.ds(c * _L, _L)]
                                 * ei_c[r, pl.ds(c * _L, _L)])
                s = jnp.sum(acc)
                w = jnp.where(lane == l, jnp.full((_L,), s), w)
            uvec = uidx_v[pl.ds(p0 + g * _L, _L)]
            ivec = iidx_v[pl.ds(p0 + g * _L, _L)]
            base16 = (lane + p0 + g * _L) * _L
            ub_vals = plsc.load_gather(ub_w, [base16 + (uvec - _win(uvec, _NU))])
            ib_vals = plsc.load_gather(ib_w, [base16 + (ivec - _win(ivec, _NI))])
            out_v[pl.ds(p0 + g * _L, _L)] = w + ub_vals + ib_vals + _MU
            return carry

        lax.fori_loop(0, _NG, group, 0)

    pltpu.sync_copy(out_v, out_hbm.at[pl.ds(base, _BPW)])


@jax.jit
def _sc_call(embed_user, embed_item, ub_flat, ib_flat, user_idx, item_idx):
    mesh = plsc.VectorSubcoreMesh(core_axis_name="c", subcore_axis_name="s")
    run = functools.partial(
        pl.kernel,
        mesh=mesh,
        compiler_params=pltpu.CompilerParams(
            needs_layout_passes=False, use_tc_tiling_on_sc=True),
        out_type=jax.ShapeDtypeStruct((_B,), jnp.float32),
        scratch_types=[
            pltpu.VMEM((_BPW,), jnp.int32),
            pltpu.VMEM((_BPW,), jnp.int32),
            pltpu.VMEM((_BPC, _D), jnp.float32),
            pltpu.VMEM((_BPC, _D), jnp.float32),
            pltpu.VMEM((_BPW * _L,), jnp.float32),
            pltpu.VMEM((_BPW * _L,), jnp.float32),
            pltpu.VMEM((_BPW,), jnp.float32),
            pltpu.SemaphoreType.DMA,
            pltpu.SemaphoreType.DMA,
        ],
    )(_sc_body)
    return run(embed_user, embed_item, ub_flat, ib_flat, user_idx, item_idx)


def kernel(embed_user, embed_item, user_bias, item_bias, user_idx, item_idx):
    ub_flat = user_bias.reshape(-1)
    ib_flat = item_bias.reshape(-1)
    return _sc_call(embed_user, embed_item, ub_flat, ib_flat,
                    user_idx.astype(jnp.int32), item_idx.astype(jnp.int32))
